# baseline (speedup 1.0000x reference)
import jax
import jax.numpy as jnp
from jax.experimental import pallas as pl
from jax.experimental.pallas import tpu as pltpu

_B, _C, _H, _W = 8, 256, 160, 160
_CR = 64
_G = 8
_HW = _H * _W
_NS = 4
_SH = _H // _NS
_SL = _HW // _NS
_PAD = 256
_EPS = 1e-4
_GN_EPS = 1e-5


def _adrc_kernel(x_ref, wr_ref, w1_ref, w2_ref, par_ref, out_ref,
                 ybf, sbuf, sums, sumsq, weff):
    p = pl.program_id(1)

    @pl.when(p < _NS)
    def _matmul_phase():
        xs = x_ref[0]
        xfb = xs.reshape(_C, _SL).astype(jnp.bfloat16)
        r = jnp.dot(wr_ref[...], xfb, preferred_element_type=jnp.float32)
        rbf = r.astype(jnp.bfloat16)
        for k in range(_NS):
            @pl.when(p == k)
            def _(k=k):
                ybf[:, _PAD + k * _SL:_PAD + (k + 1) * _SL] = rbf
        ls = jnp.sum(r, axis=1, keepdims=True)
        lq = jnp.sum(r * r, axis=1, keepdims=True)

        @pl.when(p == 0)
        def _():
            ybf[:, :_PAD] = jnp.zeros((_CR, _PAD), jnp.bfloat16)
            ybf[:, _PAD + _HW:] = jnp.zeros((_CR, _PAD), jnp.bfloat16)
            sums[...] = ls
            sumsq[...] = lq

        @pl.when(p > 0)
        def _():
            sums[...] += ls
            sumsq[...] += lq

        @pl.when(p == _NS - 1)
        def _finalize():
            npix = float((_CR // _G) * _HW)
            hi = jax.lax.Precision.HIGHEST
            r8 = jax.lax.broadcasted_iota(jnp.int32, (_G, _CR), 0)
            c8 = jax.lax.broadcasted_iota(jnp.int32, (_G, _CR), 1)
            g8 = (r8 == c8 // (_CR // _G)).astype(jnp.float32)
            r64 = jax.lax.broadcasted_iota(jnp.int32, (_CR, _G), 0)
            c64 = jax.lax.broadcasted_iota(jnp.int32, (_CR, _G), 1)
            gt = (r64 // (_CR // _G) == c64).astype(jnp.float32)
            gsum = jnp.dot(g8, sums[...], precision=hi)
            gsq = jnp.dot(g8, sumsq[...], precision=hi)
            gmean = gsum / npix
            gvar = gsq / npix - gmean * gmean
            grs = jax.lax.rsqrt(gvar + _GN_EPS)
            a_ch = jnp.dot(gt, grs, precision=hi)
            m_ch = jnp.dot(gt, gmean, precision=hi)
            a_col = par_ref[:, 0:1] * a_ch
            b_col = par_ref[:, 1:2] - m_ch * a_col
            ybf[:, _PAD:_PAD + _HW] = (
                ybf[:, _PAD:_PAD + _HW] * a_col.astype(jnp.bfloat16)
                + b_col.astype(jnp.bfloat16))
            pcol = a_col * (sums[...] / float(_HW)) + b_col
            hcol = jnp.maximum(
                jnp.dot(w1_ref[...], pcol, precision=hi) + par_ref[:16, 5:6],
                0.0)
            gam = jax.nn.sigmoid(
                jnp.dot(w2_ref[...], hcol, precision=hi) + par_ref[:, 2:3])
            weff[...] = par_ref[:, 3:4] + gam * par_ref[:, 4:5]

    @pl.when(p >= _NS)
    def _out_phase():
        for k in range(_NS):
            j = (k + _NS - 1) % _NS
            @pl.when(p == _NS + k)
            def _(j=j):
                base = _PAD + j * _SL
                sbuf[...] = ybf[:, base - _W:base + _SL + _W]

        c0 = sbuf[:, _W:_W + _SL]
        tm = sbuf[:, 0:_SL]
        tp = sbuf[:, 2 * _W:2 * _W + _SL]
        ci = jax.lax.broadcasted_iota(jnp.int32, (1, _SL), 1)
        cm = jax.lax.rem(ci, _W)
        ml = cm != 0
        mr = cm != _W - 1

        sa = tm + c0 + tp
        sb = sa + c0
        dv = tm - tp
        zc = jnp.zeros((_CR, 1), jnp.bfloat16)

        def shl(v):
            return jnp.concatenate([zc, v[:, :_SL - 1]], axis=1)

        def shr(v):
            return jnp.concatenate([v[:, 1:], zc], axis=1)

        mu9 = sa + jnp.where(ml, shl(sa), 0) + jnp.where(mr, shr(sa), 0)
        gxq = jnp.where(ml, shl(sb), 0) - jnp.where(mr, shr(sb), 0)
        gyq = jnp.where(ml, shl(dv), 0) + dv + dv + jnp.where(mr, shr(dv), 0)
        num = jnp.abs(c0 * 9.0 - mu9)
        den = jnp.abs(gxq) + jnp.abs(gyq) + 4.0 * _EPS
        ratio = jnp.minimum(num * (4.0 / 9.0) / den, 2.0)
        kap = 1.0 - ratio
        contrib = kap * weff[...].astype(jnp.bfloat16)
        asum = jnp.sum(contrib, axis=0, keepdims=True)
        sca = 1.0 + 0.1 * jax.nn.sigmoid(asum.astype(jnp.float32))
        sca2 = sca.reshape(1, _SH, _W)
        out_ref[0] = x_ref[0] * sca2


def kernel(x, reduce_w, gn_scale, gn_bias, gate_w1, gate_b1, gate_w2,
           gate_b2, fuse_w):
    wr = reduce_w.reshape(_CR, _C).astype(jnp.bfloat16)
    w1 = gate_w1.reshape(16, _CR)
    w2 = gate_w2.reshape(_CR, 16)
    fw = fuse_w.reshape(2 * _CR)
    par = jnp.stack([gn_scale, gn_bias, gate_b2, fw[:_CR], fw[_CR:],
                     jnp.pad(gate_b1, (0, _CR - 16))], axis=1)

    def x_idx(b, p):
        return (b, 0, jnp.where(p < _NS, p, jax.lax.rem(p + _NS - 1, _NS)), 0)

    def o_idx(b, p):
        return (b, 0, jnp.where(p < _NS + 1, _NS - 1,
                                jax.lax.rem(p + _NS - 1, _NS)), 0)

    return pl.pallas_call(
        _adrc_kernel,
        out_shape=jax.ShapeDtypeStruct((_B, _C, _H, _W), jnp.float32),
        grid=(_B, 2 * _NS),
        in_specs=[
            pl.BlockSpec((1, _C, _SH, _W), x_idx),
            pl.BlockSpec((_CR, _C), lambda b, p: (0, 0)),
            pl.BlockSpec((16, _CR), lambda b, p: (0, 0)),
            pl.BlockSpec((_CR, 16), lambda b, p: (0, 0)),
            pl.BlockSpec((_CR, 6), lambda b, p: (0, 0)),
        ],
        out_specs=pl.BlockSpec((1, _C, _SH, _W), o_idx),
        scratch_shapes=[
            pltpu.VMEM((_CR, _HW + 2 * _PAD), jnp.bfloat16),
            pltpu.VMEM((_CR, _SL + 2 * _W), jnp.bfloat16),
            pltpu.VMEM((_CR, 1), jnp.float32),
            pltpu.VMEM((_CR, 1), jnp.float32),
            pltpu.VMEM((_CR, 1), jnp.float32),
        ],
        compiler_params=pltpu.CompilerParams(
            dimension_semantics=("parallel", "arbitrary"),
            vmem_limit_bytes=54 * 1024 * 1024,
        ),
        name="adrc_pe_fused",
    )(x, wr, w1, w2, par)


# speedup vs baseline: 1.1536x; 1.1536x over previous
import jax
import jax.numpy as jnp
from jax.experimental import pallas as pl
from jax.experimental.pallas import tpu as pltpu

_B, _C, _H, _W = 8, 256, 160, 160
_CR = 64
_G = 8
_HW = _H * _W
_NC = 8
_CC = _C // _NC
_NS = 4
_SL = _HW // _NS
_PAD = 256
_EPS = 1e-4
_GN_EPS = 1e-5


def _adrc_kernel(x_ref, wr_ref, w1_ref, w2_ref, par_ref, out_ref,
                 ybf, xbf, sbuf, weff):
    p = pl.program_id(1)

    @pl.when(p < _NC)
    def _load_phase():
        xf = x_ref[0].reshape(_CC, _HW).astype(jnp.bfloat16)
        for k in range(_NC):
            @pl.when(p == k)
            def _(k=k):
                xbf[k * _CC:(k + 1) * _CC, :] = xf

    @pl.when(p == _NC - 1)
    def _reduce_phase():
        r = jnp.dot(wr_ref[...], xbf[...],
                    preferred_element_type=jnp.float32)
        sums = jnp.sum(r, axis=1, keepdims=True)
        sumsq = jnp.sum(r * r, axis=1, keepdims=True)
        npix = float((_CR // _G) * _HW)
        hi = jax.lax.Precision.HIGHEST
        r8 = jax.lax.broadcasted_iota(jnp.int32, (_G, _CR), 0)
        c8 = jax.lax.broadcasted_iota(jnp.int32, (_G, _CR), 1)
        g8 = (r8 == c8 // (_CR // _G)).astype(jnp.float32)
        r64 = jax.lax.broadcasted_iota(jnp.int32, (_CR, _G), 0)
        c64 = jax.lax.broadcasted_iota(jnp.int32, (_CR, _G), 1)
        gt = (r64 // (_CR // _G) == c64).astype(jnp.float32)
        gsum = jnp.dot(g8, sums, precision=hi)
        gsq = jnp.dot(g8, sumsq, precision=hi)
        gmean = gsum / npix
        gvar = gsq / npix - gmean * gmean
        grs = jax.lax.rsqrt(gvar + _GN_EPS)
        a_ch = jnp.dot(gt, grs, precision=hi)
        m_ch = jnp.dot(gt, gmean, precision=hi)
        a_col = par_ref[:, 0:1] * a_ch
        b_col = par_ref[:, 1:2] - m_ch * a_col
        ybf[:, :_PAD] = jnp.zeros((_CR, _PAD), jnp.bfloat16)
        ybf[:, _PAD + _HW:] = jnp.zeros((_CR, _PAD), jnp.bfloat16)
        ybf[:, _PAD:_PAD + _HW] = (r * a_col + b_col).astype(jnp.bfloat16)
        pcol = a_col * (sums / float(_HW)) + b_col
        hcol = jnp.maximum(
            jnp.dot(w1_ref[...], pcol, precision=hi) + par_ref[:16, 5:6],
            0.0)
        gam = jax.nn.sigmoid(
            jnp.dot(w2_ref[...], hcol, precision=hi) + par_ref[:, 2:3])
        weff[...] = par_ref[:, 3:4] + gam * par_ref[:, 4:5]

    @pl.when(p >= _NC)
    def _out_phase():
        for k in range(_NS):
            @pl.when(p == _NC + k)
            def _(k=k):
                base = _PAD + k * _SL
                sbuf[...] = ybf[:, base - _W:base + _SL + _W]

        c0 = sbuf[:, _W:_W + _SL]
        tm = sbuf[:, 0:_SL]
        tp = sbuf[:, 2 * _W:2 * _W + _SL]
        ci = jax.lax.broadcasted_iota(jnp.int32, (1, _SL), 1)
        cm = jax.lax.rem(ci, _W)
        ml = cm != 0
        mr = cm != _W - 1

        sa = tm + c0 + tp
        sb = sa + c0
        dv = tm - tp
        zc = jnp.zeros((_CR, 1), jnp.bfloat16)

        def shl(v):
            return jnp.concatenate([zc, v[:, :_SL - 1]], axis=1)

        def shr(v):
            return jnp.concatenate([v[:, 1:], zc], axis=1)

        mu9 = sa + jnp.where(ml, shl(sa), 0) + jnp.where(mr, shr(sa), 0)
        gxq = jnp.where(ml, shl(sb), 0) - jnp.where(mr, shr(sb), 0)
        gyq = jnp.where(ml, shl(dv), 0) + dv + dv + jnp.where(mr, shr(dv), 0)
        num = jnp.abs(c0 * 9.0 - mu9)
        den = jnp.abs(gxq) + jnp.abs(gyq) + 4.0 * _EPS
        ratio = jnp.minimum(num * (4.0 / 9.0) / den, 2.0)
        kap = 1.0 - ratio
        contrib = kap * weff[...].astype(jnp.bfloat16)
        asum = jnp.sum(contrib, axis=0, keepdims=True)
        sca = 1.0 + 0.1 * jax.nn.sigmoid(asum.astype(jnp.float32))
        for k in range(_NS):
            @pl.when(p == _NC + k)
            def _(k=k):
                xv = xbf[:, k * _SL:(k + 1) * _SL].astype(jnp.float32)
                out_ref[0] = xv * sca


def kernel(x, reduce_w, gn_scale, gn_bias, gate_w1, gate_b1, gate_w2,
           gate_b2, fuse_w):
    wr = reduce_w.reshape(_CR, _C).astype(jnp.bfloat16)
    w1 = gate_w1.reshape(16, _CR)
    w2 = gate_w2.reshape(_CR, 16)
    fw = fuse_w.reshape(2 * _CR)
    par = jnp.stack([gn_scale, gn_bias, gate_b2, fw[:_CR], fw[_CR:],
                     jnp.pad(gate_b1, (0, _CR - 16))], axis=1)

    def x_idx(b, p):
        return (b, jnp.where(p < _NC, p, _NC - 1), 0, 0)

    def o_idx(b, p):
        return (b, 0, jnp.where(p < _NC + 1, 0, p - _NC))

    out3 = pl.pallas_call(
        _adrc_kernel,
        out_shape=jax.ShapeDtypeStruct((_B, _C, _HW), jnp.float32),
        grid=(_B, _NC + _NS),
        in_specs=[
            pl.BlockSpec((1, _CC, _H, _W), x_idx),
            pl.BlockSpec((_CR, _C), lambda b, p: (0, 0)),
            pl.BlockSpec((16, _CR), lambda b, p: (0, 0)),
            pl.BlockSpec((_CR, 16), lambda b, p: (0, 0)),
            pl.BlockSpec((_CR, 6), lambda b, p: (0, 0)),
        ],
        out_specs=pl.BlockSpec((1, _C, _SL), o_idx),
        scratch_shapes=[
            pltpu.VMEM((_CR, _HW + 2 * _PAD), jnp.bfloat16),
            pltpu.VMEM((_C, _HW), jnp.bfloat16),
            pltpu.VMEM((_CR, _SL + 2 * _W), jnp.bfloat16),
            pltpu.VMEM((_CR, 1), jnp.float32),
        ],
        compiler_params=pltpu.CompilerParams(
            dimension_semantics=("parallel", "arbitrary"),
            vmem_limit_bytes=54 * 1024 * 1024,
        ),
        name="adrc_pe_fused",
    )(x, wr, w1, w2, par)
    return out3.reshape(_B, _C, _H, _W)


# speedup vs baseline: 1.2359x; 1.0713x over previous
import jax
import jax.numpy as jnp
from jax.experimental import pallas as pl
from jax.experimental.pallas import tpu as pltpu

_B, _C, _H, _W = 8, 256, 160, 160
_CR = 64
_G = 8
_HW = _H * _W
_NS = 4
_SH = _H // _NS
_SL = _HW // _NS
_PAD = 256
_EPS = 1e-4
_GN_EPS = 1e-5


def _adrc_kernel(x_ref, wr_ref, w1_ref, w2_ref, par_ref, out_ref,
                 ybf, sbuf, sums, sumsq, weff):
    p = pl.program_id(1)

    @pl.when(p < _NS)
    def _matmul_phase():
        xs = x_ref[0]
        r = jnp.dot(wr_ref[...], xs, preferred_element_type=jnp.float32)
        rbf = r.astype(jnp.bfloat16)
        for k in range(_NS):
            @pl.when(p == k)
            def _(k=k):
                ybf[:, _PAD + k * _SL:_PAD + (k + 1) * _SL] = rbf
        ls = jnp.sum(r, axis=1, keepdims=True)
        lq = jnp.sum(r * r, axis=1, keepdims=True)

        @pl.when(p == 0)
        def _():
            ybf[:, :_PAD] = jnp.zeros((_CR, _PAD), jnp.bfloat16)
            ybf[:, _PAD + _HW:] = jnp.zeros((_CR, _PAD), jnp.bfloat16)
            sums[...] = ls
            sumsq[...] = lq

        @pl.when(p > 0)
        def _():
            sums[...] += ls
            sumsq[...] += lq

        @pl.when(p == _NS - 1)
        def _finalize():
            npix = float((_CR // _G) * _HW)
            hi = jax.lax.Precision.HIGHEST
            r8 = jax.lax.broadcasted_iota(jnp.int32, (_G, _CR), 0)
            c8 = jax.lax.broadcasted_iota(jnp.int32, (_G, _CR), 1)
            g8 = (r8 == c8 // (_CR // _G)).astype(jnp.float32)
            r64 = jax.lax.broadcasted_iota(jnp.int32, (_CR, _G), 0)
            c64 = jax.lax.broadcasted_iota(jnp.int32, (_CR, _G), 1)
            gt = (r64 // (_CR // _G) == c64).astype(jnp.float32)
            gsum = jnp.dot(g8, sums[...], precision=hi)
            gsq = jnp.dot(g8, sumsq[...], precision=hi)
            gmean = gsum / npix
            gvar = gsq / npix - gmean * gmean
            grs = jax.lax.rsqrt(gvar + _GN_EPS)
            a_ch = jnp.dot(gt, grs, precision=hi)
            m_ch = jnp.dot(gt, gmean, precision=hi)
            a_col = par_ref[:, 0:1] * a_ch
            b_col = par_ref[:, 1:2] - m_ch * a_col
            ybf[:, _PAD:_PAD + _HW] = (
                ybf[:, _PAD:_PAD + _HW] * a_col.astype(jnp.bfloat16)
                + b_col.astype(jnp.bfloat16))
            pcol = a_col * (sums[...] / float(_HW)) + b_col
            hcol = jnp.maximum(
                jnp.dot(w1_ref[...], pcol, precision=hi) + par_ref[:16, 5:6],
                0.0)
            gam = jax.nn.sigmoid(
                jnp.dot(w2_ref[...], hcol, precision=hi) + par_ref[:, 2:3])
            weff[...] = par_ref[:, 3:4] + gam * par_ref[:, 4:5]

    @pl.when(p >= _NS)
    def _out_phase():
        for k in range(_NS):
            j = (k + _NS - 1) % _NS
            @pl.when(p == _NS + k)
            def _(j=j):
                base = _PAD + j * _SL
                sbuf[...] = ybf[:, base - _W:base + _SL + _W]

        c0 = sbuf[:, _W:_W + _SL]
        tm = sbuf[:, 0:_SL]
        tp = sbuf[:, 2 * _W:2 * _W + _SL]
        ci = jax.lax.broadcasted_iota(jnp.int32, (1, _SL), 1)
        cm = jax.lax.rem(ci, _W)
        ml = cm != 0
        mr = cm != _W - 1

        sa = tm + c0 + tp
        sb = sa + c0
        dv = tm - tp
        zc = jnp.zeros((_CR, 1), jnp.bfloat16)

        def shl(v):
            return jnp.concatenate([zc, v[:, :_SL - 1]], axis=1)

        def shr(v):
            return jnp.concatenate([v[:, 1:], zc], axis=1)

        mu9 = sa + jnp.where(ml, shl(sa), 0) + jnp.where(mr, shr(sa), 0)
        gxq = jnp.where(ml, shl(sb), 0) - jnp.where(mr, shr(sb), 0)
        gyq = jnp.where(ml, shl(dv), 0) + dv + dv + jnp.where(mr, shr(dv), 0)
        num = jnp.abs(c0 * 9.0 - mu9)
        den = jnp.abs(gxq) + jnp.abs(gyq) + 4.0 * _EPS
        ratio = jnp.minimum(num * (4.0 / 9.0) / den, 2.0)
        kap = 1.0 - ratio
        contrib = kap * weff[...].astype(jnp.bfloat16)
        asum = jnp.sum(contrib, axis=0, keepdims=True)
        sca = 1.0 + 0.1 * jax.nn.sigmoid(asum.astype(jnp.float32))
        prod = x_ref[0] * sca
        out_ref[0] = prod.reshape(_C, _SH, _W)


def kernel(x, reduce_w, gn_scale, gn_bias, gate_w1, gate_b1, gate_w2,
           gate_b2, fuse_w):
    x3 = x.reshape(_B, _C, _HW)
    wr = reduce_w.reshape(_CR, _C)
    w1 = gate_w1.reshape(16, _CR)
    w2 = gate_w2.reshape(_CR, 16)
    fw = fuse_w.reshape(2 * _CR)
    par = jnp.stack([gn_scale, gn_bias, gate_b2, fw[:_CR], fw[_CR:],
                     jnp.pad(gate_b1, (0, _CR - 16))], axis=1)

    def x_idx(b, p):
        return (b, 0, jnp.where(p < _NS, p, jax.lax.rem(p + _NS - 1, _NS)))

    def o_idx(b, p):
        return (b, 0, jnp.where(p < _NS + 1, _NS - 1,
                                jax.lax.rem(p + _NS - 1, _NS)), 0)

    return pl.pallas_call(
        _adrc_kernel,
        out_shape=jax.ShapeDtypeStruct((_B, _C, _H, _W), jnp.float32),
        grid=(_B, 2 * _NS),
        in_specs=[
            pl.BlockSpec((1, _C, _SL), x_idx),
            pl.BlockSpec((_CR, _C), lambda b, p: (0, 0)),
            pl.BlockSpec((16, _CR), lambda b, p: (0, 0)),
            pl.BlockSpec((_CR, 16), lambda b, p: (0, 0)),
            pl.BlockSpec((_CR, 6), lambda b, p: (0, 0)),
        ],
        out_specs=pl.BlockSpec((1, _C, _SH, _W), o_idx),
        scratch_shapes=[
            pltpu.VMEM((_CR, _HW + 2 * _PAD), jnp.bfloat16),
            pltpu.VMEM((_CR, _SL + 2 * _W), jnp.bfloat16),
            pltpu.VMEM((_CR, 1), jnp.float32),
            pltpu.VMEM((_CR, 1), jnp.float32),
            pltpu.VMEM((_CR, 1), jnp.float32),
        ],
        compiler_params=pltpu.CompilerParams(
            dimension_semantics=("parallel", "arbitrary"),
            vmem_limit_bytes=54 * 1024 * 1024,
        ),
        name="adrc_pe_fused",
    )(x3, wr, w1, w2, par)


# speedup vs baseline: 1.6024x; 1.2965x over previous
import jax
import jax.numpy as jnp
from jax.experimental import pallas as pl
from jax.experimental.pallas import tpu as pltpu

_B, _C, _H, _W = 8, 256, 160, 160
_CR = 64
_G = 8
_HW = _H * _W
_NS = 4
_SL = _HW // _NS
_PAD = 256
_EPS = 1e-4
_GN_EPS = 1e-5


def _adrc_kernel(x_ref, wr_ref, w1_ref, w2_ref, par_ref, out_ref,
                 ybf, sbuf, sums, sumsq, weff):
    p = pl.program_id(1)

    @pl.when(p < _NS)
    def _matmul_phase():
        xs = x_ref[0]
        r = jnp.dot(wr_ref[...], xs, preferred_element_type=jnp.float32)
        rbf = r.astype(jnp.bfloat16)
        for k in range(_NS):
            @pl.when(p == k)
            def _(k=k):
                ybf[:, _PAD + k * _SL:_PAD + (k + 1) * _SL] = rbf
        ls = jnp.sum(r, axis=1, keepdims=True)
        lq = jnp.sum(r * r, axis=1, keepdims=True)

        @pl.when(p == 0)
        def _():
            ybf[:, :_PAD] = jnp.zeros((_CR, _PAD), jnp.bfloat16)
            ybf[:, _PAD + _HW:] = jnp.zeros((_CR, _PAD), jnp.bfloat16)
            sums[...] = ls
            sumsq[...] = lq

        @pl.when(p > 0)
        def _():
            sums[...] += ls
            sumsq[...] += lq

        @pl.when(p == _NS - 1)
        def _finalize():
            npix = float((_CR // _G) * _HW)
            hi = jax.lax.Precision.HIGHEST
            r8 = jax.lax.broadcasted_iota(jnp.int32, (_G, _CR), 0)
            c8 = jax.lax.broadcasted_iota(jnp.int32, (_G, _CR), 1)
            g8 = (r8 == c8 // (_CR // _G)).astype(jnp.float32)
            r64 = jax.lax.broadcasted_iota(jnp.int32, (_CR, _G), 0)
            c64 = jax.lax.broadcasted_iota(jnp.int32, (_CR, _G), 1)
            gt = (r64 // (_CR // _G) == c64).astype(jnp.float32)
            gsum = jnp.dot(g8, sums[...], precision=hi)
            gsq = jnp.dot(g8, sumsq[...], precision=hi)
            gmean = gsum / npix
            gvar = gsq / npix - gmean * gmean
            grs = jax.lax.rsqrt(gvar + _GN_EPS)
            a_ch = jnp.dot(gt, grs, precision=hi)
            m_ch = jnp.dot(gt, gmean, precision=hi)
            a_col = par_ref[:, 0:1] * a_ch
            b_col = par_ref[:, 1:2] - m_ch * a_col
            ybf[:, _PAD:_PAD + _HW] = (
                ybf[:, _PAD:_PAD + _HW] * a_col.astype(jnp.bfloat16)
                + b_col.astype(jnp.bfloat16))
            pcol = a_col * (sums[...] / float(_HW)) + b_col
            hcol = jnp.maximum(
                jnp.dot(w1_ref[...], pcol, precision=hi) + par_ref[:16, 5:6],
                0.0)
            gam = jax.nn.sigmoid(
                jnp.dot(w2_ref[...], hcol, precision=hi) + par_ref[:, 2:3])
            weff[...] = par_ref[:, 3:4] + gam * par_ref[:, 4:5]

    @pl.when(p >= _NS)
    def _out_phase():
        for k in range(_NS):
            j = (k + _NS - 1) % _NS
            @pl.when(p == _NS + k)
            def _(j=j):
                base = _PAD + j * _SL
                sbuf[...] = ybf[:, base - _W:base + _SL + _W]

        c0 = sbuf[:, _W:_W + _SL]
        tm = sbuf[:, 0:_SL]
        tp = sbuf[:, 2 * _W:2 * _W + _SL]
        ci = jax.lax.broadcasted_iota(jnp.int32, (1, _SL), 1)
        cm = jax.lax.rem(ci, _W)
        ml = cm != 0
        mr = cm != _W - 1

        sa = tm + c0 + tp
        sb = sa + c0
        dv = tm - tp
        zc = jnp.zeros((_CR, 1), jnp.bfloat16)

        def shl(v):
            return jnp.concatenate([zc, v[:, :_SL - 1]], axis=1)

        def shr(v):
            return jnp.concatenate([v[:, 1:], zc], axis=1)

        mu9 = sa + jnp.where(ml, shl(sa), 0) + jnp.where(mr, shr(sa), 0)
        gxq = jnp.where(ml, shl(sb), 0) - jnp.where(mr, shr(sb), 0)
        gyq = jnp.where(ml, shl(dv), 0) + dv + dv + jnp.where(mr, shr(dv), 0)
        num = jnp.abs(c0 * 9.0 - mu9)
        den = jnp.abs(gxq) + jnp.abs(gyq) + 4.0 * _EPS
        ratio = jnp.minimum(num * (4.0 / 9.0) / den, 2.0)
        kap = 1.0 - ratio
        contrib = kap * weff[...].astype(jnp.bfloat16)
        asum = jnp.sum(contrib, axis=0, keepdims=True)
        sca = 1.0 + 0.1 * jax.nn.sigmoid(asum.astype(jnp.float32))
        out_ref[0] = x_ref[0] * sca


def kernel(x, reduce_w, gn_scale, gn_bias, gate_w1, gate_b1, gate_w2,
           gate_b2, fuse_w):
    x3 = x.reshape(_B, _C, _HW)
    wr = reduce_w.reshape(_CR, _C)
    w1 = gate_w1.reshape(16, _CR)
    w2 = gate_w2.reshape(_CR, 16)
    fw = fuse_w.reshape(2 * _CR)
    par = jnp.stack([gn_scale, gn_bias, gate_b2, fw[:_CR], fw[_CR:],
                     jnp.pad(gate_b1, (0, _CR - 16))], axis=1)

    def x_idx(b, p):
        return (b, 0, jnp.where(p < _NS, p, jax.lax.rem(p + _NS - 1, _NS)))

    def o_idx(b, p):
        return (b, 0, jnp.where(p < _NS + 1, _NS - 1,
                                jax.lax.rem(p + _NS - 1, _NS)))

    out3 = pl.pallas_call(
        _adrc_kernel,
        out_shape=jax.ShapeDtypeStruct((_B, _C, _HW), jnp.float32),
        grid=(_B, 2 * _NS),
        in_specs=[
            pl.BlockSpec((1, _C, _SL), x_idx),
            pl.BlockSpec((_CR, _C), lambda b, p: (0, 0)),
            pl.BlockSpec((16, _CR), lambda b, p: (0, 0)),
            pl.BlockSpec((_CR, 16), lambda b, p: (0, 0)),
            pl.BlockSpec((_CR, 6), lambda b, p: (0, 0)),
        ],
        out_specs=pl.BlockSpec((1, _C, _SL), o_idx),
        scratch_shapes=[
            pltpu.VMEM((_CR, _HW + 2 * _PAD), jnp.bfloat16),
            pltpu.VMEM((_CR, _SL + 2 * _W), jnp.bfloat16),
            pltpu.VMEM((_CR, 1), jnp.float32),
            pltpu.VMEM((_CR, 1), jnp.float32),
            pltpu.VMEM((_CR, 1), jnp.float32),
        ],
        compiler_params=pltpu.CompilerParams(
            dimension_semantics=("parallel", "arbitrary"),
            vmem_limit_bytes=52 * 1024 * 1024,
        ),
        name="adrc_pe_fused",
    )(x3, wr, w1, w2, par)
    return out3.reshape(_B, _C, _H, _W)
